# baseline (speedup 1.0000x reference)
import jax
import jax.numpy as jnp
from jax import lax
from jax.experimental import pallas as pl
from jax.experimental.pallas import tpu as pltpu, tpu_sc as plsc

N_VOCAB = 1000000
N_EMBED = 64
N_TOKENS = 200
BATCH = 4096

NC, NS = 2, 16
NW = NC * NS
HALF = 32
ROWS32 = BATCH * N_TOKENS * 2
ROWS32_PER_W = ROWS32 // NW
CHUNK = 128
CHUNKS_PER_W = ROWS32_PER_W // CHUNK


def _body(idx_hbm, pid_hbm, table_hbm, pos_hbm, out_hbm,
          idx_v, pid_v, rows_v, sem):
    wid = lax.axis_index("s") * NC + lax.axis_index("c")
    w_base = wid * ROWS32_PER_W

    def chunk_step(c, carry):
        base = w_base + c * CHUNK
        pltpu.sync_copy(idx_hbm.at[pl.ds(base, CHUNK)], idx_v)
        pltpu.sync_copy(pid_hbm.at[pl.ds(base, CHUNK)], pid_v)
        pltpu.async_copy(table_hbm.at[idx_v], rows_v, sem).wait()
        pltpu.async_copy(pos_hbm.at[pid_v], rows_v, sem, add=True).wait()
        pltpu.sync_copy(rows_v, out_hbm.at[pl.ds(base, CHUNK)])
        return carry

    lax.fori_loop(0, CHUNKS_PER_W, chunk_step, 0)


@jax.jit
def kernel(tokens, token_table, position_embedding):
    tok_flat = tokens.reshape(-1).astype(jnp.int32)
    idx32 = (tok_flat[:, None] * 2 + jnp.arange(2, dtype=jnp.int32)).reshape(-1)
    pid32 = jnp.broadcast_to(
        jnp.arange(2 * N_TOKENS, dtype=jnp.int32), (BATCH, 2 * N_TOKENS)
    ).reshape(-1)
    table32 = token_table.reshape(2 * N_VOCAB, HALF)
    pos32 = position_embedding.reshape(2 * N_TOKENS, HALF)
    mesh = plsc.VectorSubcoreMesh(core_axis_name="c", subcore_axis_name="s")
    out32 = pl.kernel(
        _body,
        out_type=jax.ShapeDtypeStruct((ROWS32, HALF), jnp.float32),
        mesh=mesh,
        compiler_params=pltpu.CompilerParams(use_tc_tiling_on_sc=False),
        scratch_types=[
            pltpu.VMEM((CHUNK,), jnp.int32),
            pltpu.VMEM((CHUNK,), jnp.int32),
            pltpu.VMEM((CHUNK, HALF), jnp.float32),
            pltpu.SemaphoreType.DMA,
        ],
    )(idx32, pid32, table32, pos32)
    return out32.reshape(BATCH, N_TOKENS, N_EMBED)


# speedup vs baseline: 1.5089x; 1.5089x over previous
import jax
import jax.numpy as jnp
from jax import lax
from jax.experimental import pallas as pl
from jax.experimental.pallas import tpu as pltpu, tpu_sc as plsc

N_VOCAB = 1000000
N_EMBED = 64
N_TOKENS = 200
BATCH = 4096

NC, NS = 2, 16
NW = NC * NS
HALF = 32
ROWS32 = BATCH * N_TOKENS * 2
ROWS32_PER_W = ROWS32 // NW
CHUNK = 128
CHUNKS_PER_W = ROWS32_PER_W // CHUNK
NBUF = 4
GROUPS = CHUNKS_PER_W // NBUF


def _body(idx_hbm, pid_hbm, table_hbm, pos_hbm, out_hbm,
          idx_all, pid_all, rows, sems):
    wid = lax.axis_index("s") * NC + lax.axis_index("c")
    w_cbase = wid * CHUNKS_PER_W
    w_base = wid * ROWS32_PER_W

    pltpu.sync_copy(idx_hbm.at[pl.ds(w_cbase, CHUNKS_PER_W)], idx_all)
    pltpu.sync_copy(pid_hbm.at[pl.ds(w_cbase, CHUNKS_PER_W)], pid_all)

    def gather_start(c, b):
        return pltpu.async_copy(table_hbm.at[idx_all.at[c]], rows.at[b], sems[b])

    def add_start(c, b):
        return pltpu.async_copy(pos_hbm.at[pid_all.at[c]], rows.at[b], sems[b],
                                add=True)

    def store_start(c, b):
        base = w_base + c * CHUNK
        return pltpu.async_copy(rows.at[b], out_hbm.at[pl.ds(base, CHUNK)],
                                sems[b])

    def store_wait(b):
        pltpu.make_async_copy(rows.at[b], out_hbm.at[pl.ds(0, CHUNK)],
                              sems[b]).wait()

    def group_step(g, carry):
        c0 = g * NBUF
        for b in range(NBUF):
            @pl.when(g > 0)
            def _():
                store_wait(b)
            gather_start(c0 + b, b)
        for b in range(NBUF):
            pltpu.make_async_copy(table_hbm.at[idx_all.at[0]], rows.at[b],
                                  sems[b]).wait()
            add_start(c0 + b, b)
        for b in range(NBUF):
            pltpu.make_async_copy(pos_hbm.at[pid_all.at[0]], rows.at[b],
                                  sems[b]).wait()
            store_start(c0 + b, b)
        return carry

    lax.fori_loop(0, GROUPS, group_step, 0, unroll=False)
    for b in range(NBUF):
        store_wait(b)


@jax.jit
def kernel(tokens, token_table, position_embedding):
    tok_flat = tokens.reshape(-1).astype(jnp.int32)
    idx32 = ((tok_flat[:, None] * 2 + jnp.arange(2, dtype=jnp.int32))
             .reshape(ROWS32 // CHUNK, CHUNK))
    pid32 = jnp.broadcast_to(
        jnp.arange(2 * N_TOKENS, dtype=jnp.int32), (BATCH, 2 * N_TOKENS)
    ).reshape(ROWS32 // CHUNK, CHUNK)
    table32 = token_table.reshape(2 * N_VOCAB, HALF)
    pos32 = position_embedding.reshape(2 * N_TOKENS, HALF)
    mesh = plsc.VectorSubcoreMesh(core_axis_name="c", subcore_axis_name="s")
    out32 = pl.kernel(
        _body,
        out_type=jax.ShapeDtypeStruct((ROWS32, HALF), jnp.float32),
        mesh=mesh,
        compiler_params=pltpu.CompilerParams(use_tc_tiling_on_sc=False),
        scratch_types=[
            pltpu.VMEM((CHUNKS_PER_W, CHUNK), jnp.int32),
            pltpu.VMEM((CHUNKS_PER_W, CHUNK), jnp.int32),
            pltpu.VMEM((NBUF, CHUNK, HALF), jnp.float32),
            [pltpu.SemaphoreType.DMA] * NBUF,
        ],
    )(idx32, pid32, table32, pos32)
    return out32.reshape(BATCH, N_TOKENS, N_EMBED)


# speedup vs baseline: 1.5248x; 1.0106x over previous
import jax
import jax.numpy as jnp
from jax import lax
from jax.experimental import pallas as pl
from jax.experimental.pallas import tpu as pltpu, tpu_sc as plsc

N_VOCAB = 1000000
N_EMBED = 64
N_TOKENS = 200
BATCH = 4096

NC, NS = 2, 16
NW = NC * NS
ROWS = BATCH * N_TOKENS
ROWS_PER_W = ROWS // NW
CHUNK = 128
CHUNKS_PER_W = ROWS_PER_W // CHUNK
NBUF = 8
GROUPS = CHUNKS_PER_W // NBUF


def _body(tok_hbm, pid_hbm, table_hbm, pos_hbm, out_hbm,
          idx_ring, pid_ring, rows, sems):
    wid = lax.axis_index("s") * NC + lax.axis_index("c")
    w_base = wid * ROWS_PER_W

    def idx_start(c, b):
        base = w_base + c * CHUNK
        pltpu.async_copy(tok_hbm.at[pl.ds(base, CHUNK)], idx_ring.at[b], sems[b])
        pltpu.async_copy(pid_hbm.at[pl.ds(base, CHUNK)], pid_ring.at[b], sems[b])

    def idx_wait(b):
        pltpu.make_async_copy(tok_hbm.at[pl.ds(0, CHUNK)], idx_ring.at[b],
                              sems[b]).wait()
        pltpu.make_async_copy(pid_hbm.at[pl.ds(0, CHUNK)], pid_ring.at[b],
                              sems[b]).wait()

    def rows_wait(b):
        pltpu.make_async_copy(rows.at[b], out_hbm.at[pl.ds(0, CHUNK)],
                              sems[b]).wait()

    def group_step(g, carry):
        c0 = g * NBUF
        for b in range(NBUF):
            @pl.when(g > 0)
            def _():
                rows_wait(b)
            idx_start(c0 + b, b)
        for b in range(NBUF):
            idx_wait(b)
            pltpu.async_copy(table_hbm.at[idx_ring.at[b]], rows.at[b], sems[b])
        for b in range(NBUF):
            rows_wait(b)
            pltpu.async_copy(pos_hbm.at[pid_ring.at[b]], rows.at[b], sems[b],
                             add=True)
        for b in range(NBUF):
            rows_wait(b)
            base = w_base + (c0 + b) * CHUNK
            pltpu.async_copy(rows.at[b], out_hbm.at[pl.ds(base, CHUNK)], sems[b])
        return carry

    lax.fori_loop(0, GROUPS, group_step, 0, unroll=False)
    for b in range(NBUF):
        rows_wait(b)


@jax.jit
def kernel(tokens, token_table, position_embedding):
    tok_flat = tokens.reshape(-1).astype(jnp.int32)
    pid = jnp.broadcast_to(
        jnp.arange(N_TOKENS, dtype=jnp.int32), (BATCH, N_TOKENS)
    ).reshape(-1)
    mesh = plsc.VectorSubcoreMesh(core_axis_name="c", subcore_axis_name="s")
    out = pl.kernel(
        _body,
        out_type=jax.ShapeDtypeStruct((ROWS, N_EMBED), jnp.float32),
        mesh=mesh,
        compiler_params=pltpu.CompilerParams(use_tc_tiling_on_sc=False),
        scratch_types=[
            pltpu.VMEM((NBUF, CHUNK), jnp.int32),
            pltpu.VMEM((NBUF, CHUNK), jnp.int32),
            pltpu.VMEM((NBUF, CHUNK, N_EMBED), jnp.float32),
            [pltpu.SemaphoreType.DMA] * NBUF,
        ],
    )(tok_flat, pid, token_table, position_embedding)
    return out.reshape(BATCH, N_TOKENS, N_EMBED)


# speedup vs baseline: 1.5570x; 1.0211x over previous
import jax
import jax.numpy as jnp
from jax import lax
from jax.experimental import pallas as pl
from jax.experimental.pallas import tpu as pltpu, tpu_sc as plsc

N_VOCAB = 1000000
N_EMBED = 64
N_TOKENS = 200
BATCH = 4096

NC, NS = 2, 16
NW = NC * NS
ROWS = BATCH * N_TOKENS
ROWS_PER_W = ROWS // NW
CHUNK = 128
CHUNKS_PER_W = ROWS_PER_W // CHUNK
NBUF = 8
GROUPS = CHUNKS_PER_W // NBUF
POS_EXT = N_TOKENS + CHUNK
LANES = 16
VPR = N_EMBED // LANES


def _body(tok_hbm, table_hbm, posx_hbm, out_hbm,
          idx_ring, pos_v, rows, isems, gsems, ssems):
    wid = lax.axis_index("s") * NC + lax.axis_index("c")
    w_base = wid * ROWS_PER_W

    pltpu.sync_copy(posx_hbm, pos_v)

    def idx_start(c, b):
        base = w_base + c * CHUNK
        pltpu.async_copy(tok_hbm.at[pl.ds(base, CHUNK)], idx_ring.at[b],
                         isems[b])

    def idx_wait(b):
        pltpu.make_async_copy(tok_hbm.at[pl.ds(0, CHUNK)], idx_ring.at[b],
                              isems[b]).wait()

    def gather_start(b):
        pltpu.async_copy(table_hbm.at[idx_ring.at[b]], rows.at[b], gsems[b])

    def gather_wait(b):
        pltpu.make_async_copy(table_hbm.at[idx_ring.at[b]], rows.at[b],
                              gsems[b]).wait()

    def store_start(c, b):
        base = w_base + c * CHUNK
        pltpu.async_copy(rows.at[b], out_hbm.at[pl.ds(base, CHUNK)], ssems[b])

    def store_wait(b):
        pltpu.make_async_copy(rows.at[b], out_hbm.at[pl.ds(0, CHUNK)],
                              ssems[b]).wait()

    def pos_add(c, b):
        phase = (c * CHUNK) % N_TOKENS

        def row_step(i, carry):
            for j in range(VPR):
                sl = pl.ds(j * LANES, LANES)
                rows.at[b][i, sl] += pos_v[phase + i, sl]
            return carry

        lax.fori_loop(0, CHUNK, row_step, 0, unroll=4)

    def group_step(g, carry):
        c0 = g * NBUF
        for b in range(NBUF):
            @pl.when(g > 0)
            def _():
                store_wait(b)
            idx_start(c0 + b, b)
        for b in range(NBUF):
            idx_wait(b)
            gather_start(b)
        for b in range(NBUF):
            gather_wait(b)
            pos_add(c0 + b, b)
        for b in range(NBUF):
            store_start(c0 + b, b)
        return carry

    lax.fori_loop(0, GROUPS, group_step, 0, unroll=False)
    for b in range(NBUF):
        store_wait(b)


@jax.jit
def kernel(tokens, token_table, position_embedding):
    tok_flat = tokens.reshape(-1).astype(jnp.int32)
    pos_ext = jnp.concatenate(
        [position_embedding, position_embedding[:CHUNK]], axis=0)
    mesh = plsc.VectorSubcoreMesh(core_axis_name="c", subcore_axis_name="s")
    out = pl.kernel(
        _body,
        out_type=jax.ShapeDtypeStruct((ROWS, N_EMBED), jnp.float32),
        mesh=mesh,
        compiler_params=pltpu.CompilerParams(use_tc_tiling_on_sc=False),
        scratch_types=[
            pltpu.VMEM((NBUF, CHUNK), jnp.int32),
            pltpu.VMEM((POS_EXT, N_EMBED), jnp.float32),
            pltpu.VMEM((NBUF, CHUNK, N_EMBED), jnp.float32),
            [pltpu.SemaphoreType.DMA] * NBUF,
            [pltpu.SemaphoreType.DMA] * NBUF,
            [pltpu.SemaphoreType.DMA] * NBUF,
        ],
    )(tok_flat, token_table, pos_ext)
    return out.reshape(BATCH, N_TOKENS, N_EMBED)
